# Initial kernel scaffold; baseline (speedup 1.0000x reference)
import jax
import jax.numpy as jnp
from jax.experimental import pallas as pl


def kernel(theta_0, theta_1, op_fwd, op_bwd):
    raise NotImplementedError("write your pallas kernel here")


# speedup vs baseline: 1.5577x; 1.5577x over previous
import jax
import jax.numpy as jnp
from jax import lax
from jax.experimental import pallas as pl
from jax.experimental.pallas import tpu as pltpu
from jax.experimental.pallas import tpu_sc as plsc

_N = 100000
_E = 1600000
_K = 32
_KH = 16
_NS = 16
_SUB = 128
_NSB = _E // _SUB
_R_MAIN = 6256
_R_LAST = _N - 15 * _R_MAIN


def _init_drain(cid, wid, copy_fn):
    rbase = wid * _R_MAIN

    @pl.when(wid < _NS - 1)
    def _():
        copy_fn(rbase, _R_MAIN)

    @pl.when(wid == _NS - 1)
    def _():
        copy_fn(rbase, _R_LAST)


def _sc_body(t0r, t1lo, t1hi, edge_dst, out0r, out1r, idxbuf, t0buf, t1buf, acc):
    cid = lax.axis_index("c")
    wid = lax.axis_index("s")

    def init(rbase, nrows):
        @pl.when(cid == 0)
        def _():
            pltpu.sync_copy(t1lo.at[pl.ds(rbase, nrows)],
                            acc.at[pl.ds(rbase, nrows)])

        @pl.when(cid == 1)
        def _():
            pltpu.sync_copy(t1hi.at[pl.ds(rbase, nrows)],
                            acc.at[pl.ds(rbase, nrows)])

    _init_drain(cid, wid, init)
    plsc.subcore_barrier()

    lo = (wid * _NSB) // _NS
    hi = ((wid + 1) * _NSB) // _NS

    def step(sb, carry):
        ebase = sb * _SUB
        pltpu.sync_copy(edge_dst.at[pl.ds(ebase, _SUB)], idxbuf)
        pltpu.sync_copy(t0r.at[pl.ds(ebase, _SUB), cid], t0buf)

        @pl.when(cid == 0)
        def _():
            pltpu.sync_copy(t1lo.at[idxbuf], t1buf)

        @pl.when(cid == 1)
        def _():
            pltpu.sync_copy(t1hi.at[idxbuf], t1buf)

        pltpu.sync_copy(t0buf, acc.at[idxbuf], add=True)

        @plsc.parallel_loop(0, _SUB, unroll=8)
        def _(i):
            plsc.addupdate(t1buf.at[i], t0buf[i, :])

        pltpu.sync_copy(t1buf, out0r.at[pl.ds(ebase, _SUB), cid])
        return carry

    lax.fori_loop(lo, hi, step, jnp.int32(0))

    plsc.subcore_barrier()

    def drain(rbase, nrows):
        pltpu.sync_copy(acc.at[pl.ds(rbase, nrows)],
                        out1r.at[pl.ds(rbase, nrows), cid])

    _init_drain(cid, wid, drain)


def kernel(theta_0, theta_1, op_fwd, op_bwd):
    edge_dst = op_fwd[1]
    t0r = theta_0.reshape(_E, 2, _KH)
    t1lo = theta_1[:, :_KH]
    t1hi = theta_1[:, _KH:]

    mesh = plsc.VectorSubcoreMesh(core_axis_name="c", subcore_axis_name="s")
    out0r, out1r = pl.kernel(
        _sc_body,
        out_type=[
            jax.ShapeDtypeStruct((_E, 2, _KH), jnp.float32),
            jax.ShapeDtypeStruct((_N, 2, _KH), jnp.float32),
        ],
        mesh=mesh,
        compiler_params=pltpu.CompilerParams(use_tc_tiling_on_sc=False),
        scratch_types=[
            pltpu.VMEM((_SUB,), jnp.int32),
            pltpu.VMEM((_SUB, _KH), jnp.float32),
            pltpu.VMEM((_SUB, _KH), jnp.float32),
            pltpu.VMEM_SHARED((_N, _KH), jnp.float32),
        ],
    )(t0r, t1lo, t1hi, edge_dst)

    return out0r.reshape(_E, _K), out1r.reshape(_N, _K)


# speedup vs baseline: 4.9466x; 3.1755x over previous
import jax
import jax.numpy as jnp
from jax import lax
from jax.experimental import pallas as pl
from jax.experimental.pallas import tpu as pltpu
from jax.experimental.pallas import tpu_sc as plsc

_N = 100000
_E = 1600000
_K = 32
_KH = 16
_NS = 16
_SUB = 128
_NB = _E // _SUB
_R_MAIN = 6256
_R_LAST = _N - 15 * _R_MAIN


def _sc_body(t0t, t1lo, t1hi, opf_t, out0t, out1,
             idxbuf, t0T, o0T, t0buf, t1buf, acc):
    cid = lax.axis_index("c")
    wid = lax.axis_index("s")

    iota = lax.iota(jnp.int32, 16)
    gg_idx = lax.shift_right_logical(iota, 3)
    r8_idx = lax.bitwise_and(iota, 7)

    def for_my_half(fn):
        @pl.when(cid == 0)
        def _():
            fn(t1lo, 0, 0)

        @pl.when(cid == 1)
        def _():
            fn(t1hi, _KH, 2)

    def per_tile_rows(fn):
        rbase = wid * _R_MAIN

        @pl.when(wid < _NS - 1)
        def _():
            fn(rbase, _R_MAIN)

        @pl.when(wid == _NS - 1)
        def _():
            fn(rbase, _R_LAST)

    def init(t1h, koff, g0):
        def cp(rbase, nrows):
            pltpu.sync_copy(t1h.at[pl.ds(rbase, nrows)],
                            acc.at[pl.ds(rbase, nrows)])
        per_tile_rows(cp)

    for_my_half(init)
    plsc.subcore_barrier()

    lo = (wid * _NB) // _NS
    hi = ((wid + 1) * _NB) // _NS

    def run_half(t1h, koff, g0):
        def step(b, carry):
            pltpu.sync_copy(opf_t.at[b, 1], idxbuf)
            pltpu.sync_copy(t0t.at[pl.ds(g0, 2), b], t0T)
            pltpu.sync_copy(t1h.at[idxbuf], t1buf)

            @plsc.parallel_loop(0, _SUB, unroll=8)
            def _(c):
                csp = jnp.full((16,), c, dtype=jnp.int32)
                t0buf[c, :] = plsc.load_gather(t0T, [gg_idx, r8_idx, csp])

            pltpu.sync_copy(t0buf, acc.at[idxbuf], add=True)

            for cc in range(8):
                rows = iota + cc * 16
                for gg in range(2):
                    for r8 in range(8):
                        rsp = jnp.full((16,), gg * 8 + r8, dtype=jnp.int32)
                        g = plsc.load_gather(t1buf, [rows, rsp])
                        o0T[gg, r8, pl.ds(cc * 16, 16)] = (
                            t0T[gg, r8, pl.ds(cc * 16, 16)] + g)

            pltpu.sync_copy(o0T, out0t.at[pl.ds(g0, 2), b])
            return carry

        lax.fori_loop(lo, hi, step, jnp.int32(0))

    for_my_half(run_half)

    plsc.subcore_barrier()

    def drain(t1h, koff, g0):
        def cp(rbase, nrows):
            pltpu.sync_copy(acc.at[pl.ds(rbase, nrows)],
                            out1.at[pl.ds(rbase, nrows), pl.ds(koff, _KH)])
        per_tile_rows(cp)

    for_my_half(drain)


def kernel(theta_0, theta_1, op_fwd, op_bwd):
    t0t = jnp.transpose(theta_0.reshape(_NB, _SUB, 4, 8), (2, 0, 3, 1))
    opf_t = jnp.transpose(op_fwd.reshape(2, _NB, _SUB), (1, 0, 2))
    t1lo = theta_1[:, :_KH]
    t1hi = theta_1[:, _KH:]

    mesh = plsc.VectorSubcoreMesh(core_axis_name="c", subcore_axis_name="s")
    out0t, out1 = pl.kernel(
        _sc_body,
        out_type=[
            jax.ShapeDtypeStruct((4, _NB, 8, _SUB), jnp.float32),
            jax.ShapeDtypeStruct((_N, _K), jnp.float32),
        ],
        mesh=mesh,
        compiler_params=pltpu.CompilerParams(
            use_tc_tiling_on_sc=False, needs_layout_passes=False),
        scratch_types=[
            pltpu.VMEM((_SUB,), jnp.int32),
            pltpu.VMEM((2, 8, _SUB), jnp.float32),
            pltpu.VMEM((2, 8, _SUB), jnp.float32),
            pltpu.VMEM((_SUB, _KH), jnp.float32),
            pltpu.VMEM((_SUB, _KH), jnp.float32),
            pltpu.VMEM_SHARED((_N, _KH), jnp.float32),
        ],
    )(t0t, t1lo, t1hi, opf_t)

    out0 = jnp.transpose(out0t, (1, 3, 0, 2)).reshape(_E, _K)
    return out0, out1


# speedup vs baseline: 8.7398x; 1.7668x over previous
import jax
import jax.numpy as jnp
from jax import lax
from jax.experimental import pallas as pl
from jax.experimental.pallas import tpu as pltpu
from jax.experimental.pallas import tpu_sc as plsc

_N = 100000
_E = 1600000
_K = 32
_KH = 16
_NS = 16
_SUB = 128
_NB = _E // _SUB
_CH = 2
_NIT = _NB // _CH
_R_MAIN = 6256
_R_LAST = _N - 15 * _R_MAIN


def _sc_body(t0t, t1lo, t1hi, opf_t, out0t, out1,
             idxb, t0Ts, t0b, t1b, acc, sem_in, sem_g, sem_sc, sem_out):
    cid = lax.axis_index("c")
    wid = lax.axis_index("s")

    iota = lax.iota(jnp.int32, 16)
    gg_idx = lax.shift_right_logical(iota, 3)
    r8_idx = lax.bitwise_and(iota, 7)

    def for_my_half(fn):
        @pl.when(cid == 0)
        def _():
            fn(t1lo, 0, 0)

        @pl.when(cid == 1)
        def _():
            fn(t1hi, _KH, 2)

    def per_tile_rows(fn):
        rbase = wid * _R_MAIN

        @pl.when(wid < _NS - 1)
        def _():
            fn(rbase, _R_MAIN)

        @pl.when(wid == _NS - 1)
        def _():
            fn(rbase, _R_LAST)

    def init(t1h, koff, g0):
        def cp(rbase, nrows):
            pltpu.sync_copy(t1h.at[pl.ds(rbase, nrows)],
                            acc.at[pl.ds(rbase, nrows)])
        per_tile_rows(cp)

    for_my_half(init)
    plsc.subcore_barrier()

    lo = (wid * _NIT) // _NS
    hi = ((wid + 1) * _NIT) // _NS

    def run_half(t1h, koff, g0):
        def issue_in(it, s):
            b = it * _CH
            pltpu.async_copy(opf_t.at[pl.ds(b, _CH)], idxb.at[s], sem_in)
            pltpu.async_copy(t0t.at[pl.ds(g0, 2), pl.ds(b, _CH)],
                             t0Ts.at[s], sem_in)

        def wait_in():
            pltpu.make_async_copy(opf_t.at[pl.ds(0, _CH)], idxb.at[0],
                                  sem_in).wait()
            pltpu.make_async_copy(t0t.at[pl.ds(0, 2), pl.ds(0, _CH)],
                                  t0Ts.at[0], sem_in).wait()

        def wait_gather():
            for j in range(_CH):
                pltpu.make_async_copy(
                    t1h.at[idxb.at[0, j, 1]],
                    t1b.at[0, pl.ds(j * _SUB, _SUB)], sem_g).wait()

        def wait_scatter():
            for j in range(_CH):
                pltpu.make_async_copy(
                    t0b.at[0, pl.ds(j * _SUB, _SUB)],
                    acc.at[idxb.at[0, j, 1]], sem_sc).wait()

        def wait_out():
            pltpu.make_async_copy(t0Ts.at[0],
                                  out0t.at[pl.ds(0, 2), pl.ds(0, _CH)],
                                  sem_out).wait()

        issue_in(lo, lo & 1)

        def step(it, carry):
            s = it & 1
            b = it * _CH
            wait_in()
            for j in range(_CH):
                pltpu.async_copy(t1h.at[idxb.at[s, j, 1]],
                                 t1b.at[s, pl.ds(j * _SUB, _SUB)], sem_g)

            @pl.when(it > lo)
            def _():
                wait_scatter()
                wait_out()

            @pl.when(it + 1 < hi)
            def _():
                issue_in(it + 1, s ^ 1)

            ssp = jnp.full((16,), s, dtype=jnp.int32)
            for j in range(_CH):
                jsp = jnp.full((16,), j, dtype=jnp.int32)

                @plsc.parallel_loop(0, _SUB, unroll=8)
                def _(c):
                    csp = jnp.full((16,), c, dtype=jnp.int32)
                    t0b[s, j * _SUB + c, :] = plsc.load_gather(
                        t0Ts, [ssp, gg_idx, jsp, r8_idx, csp])

            for j in range(_CH):
                pltpu.async_copy(t0b.at[s, pl.ds(j * _SUB, _SUB)],
                                 acc.at[idxb.at[s, j, 1]], sem_sc, add=True)

            wait_gather()

            for j in range(_CH):
                jsp = jnp.full((16,), j, dtype=jnp.int32)

                @plsc.parallel_loop(0, _SUB, unroll=8)
                def _(c):
                    csp = jnp.full((16,), c, dtype=jnp.int32)
                    plsc.addupdate_scatter(
                        t0Ts, [ssp, gg_idx, jsp, r8_idx, csp],
                        t1b[s, j * _SUB + c, :])

            pltpu.async_copy(t0Ts.at[s],
                             out0t.at[pl.ds(g0, 2), pl.ds(b, _CH)], sem_out)
            return carry

        lax.fori_loop(lo, hi, step, jnp.int32(0))
        wait_scatter()
        wait_out()

    for_my_half(run_half)

    plsc.subcore_barrier()

    def drain(t1h, koff, g0):
        def cp(rbase, nrows):
            pltpu.sync_copy(acc.at[pl.ds(rbase, nrows)],
                            out1.at[pl.ds(rbase, nrows), pl.ds(koff, _KH)])
        per_tile_rows(cp)

    for_my_half(drain)


def kernel(theta_0, theta_1, op_fwd, op_bwd):
    t0t = jnp.transpose(theta_0.reshape(_NB, _SUB, 4, 8), (2, 0, 3, 1))
    opf_t = jnp.transpose(op_fwd.reshape(2, _NB, _SUB), (1, 0, 2))
    t1lo = theta_1[:, :_KH]
    t1hi = theta_1[:, _KH:]

    mesh = plsc.VectorSubcoreMesh(core_axis_name="c", subcore_axis_name="s")
    out0t, out1 = pl.kernel(
        _sc_body,
        out_type=[
            jax.ShapeDtypeStruct((4, _NB, 8, _SUB), jnp.float32),
            jax.ShapeDtypeStruct((_N, _K), jnp.float32),
        ],
        mesh=mesh,
        compiler_params=pltpu.CompilerParams(
            use_tc_tiling_on_sc=False, needs_layout_passes=False),
        scratch_types=[
            pltpu.VMEM((2, _CH, 2, _SUB), jnp.int32),
            pltpu.VMEM((2, 2, _CH, 8, _SUB), jnp.float32),
            pltpu.VMEM((2, _CH * _SUB, _KH), jnp.float32),
            pltpu.VMEM((2, _CH * _SUB, _KH), jnp.float32),
            pltpu.VMEM_SHARED((_N, _KH), jnp.float32),
            pltpu.SemaphoreType.DMA,
            pltpu.SemaphoreType.DMA,
            pltpu.SemaphoreType.DMA,
            pltpu.SemaphoreType.DMA,
        ],
    )(t0t, t1lo, t1hi, opf_t)

    out0 = jnp.transpose(out0t, (1, 3, 0, 2)).reshape(_E, _K)
    return out0, out1
